# baseline (speedup 1.0000x reference)
import functools

import jax
import jax.numpy as jnp
from jax import lax
from jax.experimental import pallas as pl
from jax.experimental.pallas import tpu as pltpu
from jax.experimental.pallas import tpu_sc as plsc

_B, _N, _V, _D = 1024, 50, 1000, 128
_BN = _B * _N
_BN_BLK = 5
_NW = 32
_CH = 80
_S = 2
_NSL = _N // _S
_TOK_SL = _BN // _S
_PW_SL = _TOK_SL // _NW
_NCH_SL = _PW_SL // _CH


def _argmax_body(x_ref, out_ref):
    blk = x_ref[...]
    m = jnp.max(blk, axis=1, keepdims=True)
    ids = lax.broadcasted_iota(jnp.int32, blk.shape, 1)
    out_ref[...] = jnp.min(jnp.where(blk == m, ids, _V), axis=1, keepdims=True)


def _argmax_slice(xt, s):
    return pl.pallas_call(
        _argmax_body,
        grid=(_NSL // _BN_BLK,),
        in_specs=[
            pl.BlockSpec(
                (_BN_BLK, _V, _B),
                lambda n, s=s: (n + s * (_NSL // _BN_BLK), 0, 0),
            )
        ],
        out_specs=pl.BlockSpec((_BN_BLK, 1, _B), lambda n: (n, 0, 0)),
        out_shape=jax.ShapeDtypeStruct((_NSL, 1, _B), jnp.int32),
    )(xt)


_mesh = plsc.VectorSubcoreMesh(core_axis_name="c", subcore_axis_name="s")


def _make_gather(slice_base):
    @functools.partial(
        pl.kernel,
        out_type=(),
        mesh=_mesh,
        scratch_types=[
            pltpu.VMEM((_PW_SL,), jnp.int32),
            pltpu.VMEM((2, _CH, _D), jnp.float32),
            pltpu.SemaphoreType.DMA,
            pltpu.SemaphoreType.DMA,
        ],
    )
    def _gather(idx_hbm, table_hbm, out_ref, idx_v, rows_v, sem0, sem1):
        wid = lax.axis_index("s") * 2 + lax.axis_index("c")
        pltpu.sync_copy(idx_hbm.at[pl.ds(wid * _PW_SL, _PW_SL)], idx_v)
        base = slice_base + wid * _PW_SL
        sems = (sem0, sem1)
        cps = [None, None]
        cps[0] = pltpu.async_copy(
            table_hbm.at[idx_v.at[pl.ds(0, _CH)]], rows_v.at[0], sem0
        )
        for c in range(_NCH_SL):
            cur = c % 2
            if c + 1 < _NCH_SL:
                cps[1 - cur] = pltpu.async_copy(
                    table_hbm.at[idx_v.at[pl.ds((c + 1) * _CH, _CH)]],
                    rows_v.at[1 - cur],
                    sems[1 - cur],
                )
            cps[cur].wait()
            pltpu.sync_copy(
                rows_v.at[cur], out_ref.at[pl.ds(base + c * _CH, _CH)]
            )

    return _gather


_gathers = tuple(_make_gather(s * _TOK_SL) for s in range(_S))


def kernel(x, table):
    xt = x.transpose(1, 2, 0)
    out_ref = jax.new_ref(jnp.zeros((_BN, _D), jnp.float32))
    for s in range(_S):
        tokens = _argmax_slice(xt, s).reshape(_TOK_SL)
        _gathers[s](tokens, table, out_ref)
    out = out_ref[...]
    return out.reshape(_N, _B, _D).transpose(1, 0, 2)


# speedup vs baseline: 1.0717x; 1.0717x over previous
import functools

import jax
import jax.numpy as jnp
from jax import lax
from jax.experimental import pallas as pl
from jax.experimental.pallas import tpu as pltpu
from jax.experimental.pallas import tpu_sc as plsc

_B, _N, _V, _D = 1024, 50, 1000, 128
_BN = _B * _N
_BN_BLK = 5
_NW = 32
_CH = 80
_N0 = 20
_N1 = _N - _N0


def _argmax_body(x_ref, out_ref):
    blk = x_ref[...]
    m = jnp.max(blk, axis=1, keepdims=True)
    ids = lax.broadcasted_iota(jnp.int32, blk.shape, 1)
    out_ref[...] = jnp.min(jnp.where(blk == m, ids, _V), axis=1, keepdims=True)


def _argmax_slice(xt, n_off, n_rows):
    return pl.pallas_call(
        _argmax_body,
        grid=(n_rows // _BN_BLK,),
        in_specs=[
            pl.BlockSpec(
                (_BN_BLK, _V, _B),
                lambda n, o=n_off // _BN_BLK: (n + o, 0, 0),
            )
        ],
        out_specs=pl.BlockSpec((_BN_BLK, 1, _B), lambda n: (n, 0, 0)),
        out_shape=jax.ShapeDtypeStruct((n_rows, 1, _B), jnp.int32),
    )(xt)


_mesh = plsc.VectorSubcoreMesh(core_axis_name="c", subcore_axis_name="s")


def _gather_loop(idx_hbm, table_hbm, out_ref, idx_v, rows_v, sem0, sem1,
                 out_base, per_w, nch):
    wid = lax.axis_index("s") * 2 + lax.axis_index("c")
    pltpu.sync_copy(idx_hbm.at[pl.ds(wid * per_w, per_w)], idx_v)
    base = out_base + wid * per_w
    sems = (sem0, sem1)
    cps = [None, None]
    cps[0] = pltpu.async_copy(
        table_hbm.at[idx_v.at[pl.ds(0, _CH)]], rows_v.at[0], sem0
    )
    for c in range(nch):
        cur = c % 2
        if c + 1 < nch:
            cps[1 - cur] = pltpu.async_copy(
                table_hbm.at[idx_v.at[pl.ds((c + 1) * _CH, _CH)]],
                rows_v.at[1 - cur],
                sems[1 - cur],
            )
        cps[cur].wait()
        pltpu.sync_copy(rows_v.at[cur], out_ref.at[pl.ds(base + c * _CH, _CH)])


def _sc_scratch(per_w):
    return [
        pltpu.VMEM((per_w,), jnp.int32),
        pltpu.VMEM((2, _CH, _D), jnp.float32),
        pltpu.SemaphoreType.DMA,
        pltpu.SemaphoreType.DMA,
    ]


_PW0 = _N0 * _B // _NW


@functools.partial(
    pl.kernel,
    out_type=jax.ShapeDtypeStruct((_BN, _D), jnp.float32),
    mesh=_mesh,
    scratch_types=_sc_scratch(_PW0),
)
def _gather0(idx_hbm, table_hbm, out_hbm, idx_v, rows_v, sem0, sem1):
    _gather_loop(idx_hbm, table_hbm, out_hbm, idx_v, rows_v, sem0, sem1,
                 0, _PW0, _PW0 // _CH)


_PW1 = _N1 * _B // _NW


@functools.partial(
    pl.kernel,
    out_type=(),
    mesh=_mesh,
    scratch_types=_sc_scratch(_PW1),
)
def _gather1(idx_hbm, table_hbm, out_ref, idx_v, rows_v, sem0, sem1):
    _gather_loop(idx_hbm, table_hbm, out_ref, idx_v, rows_v, sem0, sem1,
                 _N0 * _B, _PW1, _PW1 // _CH)


def kernel(x, table):
    xt = x.transpose(1, 2, 0)
    toks0 = _argmax_slice(xt, 0, _N0).reshape(_N0 * _B)
    out0 = _gather0(toks0, table)
    toks1 = _argmax_slice(xt, _N0, _N1).reshape(_N1 * _B)
    out_ref = jax.new_ref(out0)
    _gather1(toks1, table, out_ref)
    out = out_ref[...]
    return out.reshape(_N, _B, _D).transpose(1, 0, 2)


# speedup vs baseline: 1.1115x; 1.0372x over previous
import functools

import jax
import jax.numpy as jnp
from jax import lax
from jax.experimental import pallas as pl
from jax.experimental.pallas import tpu as pltpu
from jax.experimental.pallas import tpu_sc as plsc

_B, _N, _V, _D = 1024, 50, 1000, 128
_BN = _B * _N
_BN_BLK = 5
_NW = 32
_PER_W = _BN // _NW
_CH = 128
_CHUNKS = [(c * _CH, _CH) for c in range(_PER_W // _CH)] + [
    (_PER_W - _PER_W % _CH, _PER_W % _CH)
]
_NBUF = 3


def _argmax_body(x_ref, out_ref):
    blk = x_ref[...]
    m = jnp.max(blk, axis=1, keepdims=True)
    ids = lax.broadcasted_iota(jnp.int32, blk.shape, 1)
    out_ref[...] = jnp.min(jnp.where(blk == m, ids, _V), axis=1, keepdims=True)


def _argmax(xt):
    return pl.pallas_call(
        _argmax_body,
        grid=(_N // _BN_BLK,),
        in_specs=[pl.BlockSpec((_BN_BLK, _V, _B), lambda n: (n, 0, 0))],
        out_specs=pl.BlockSpec((_BN_BLK, 1, _B), lambda n: (n, 0, 0)),
        out_shape=jax.ShapeDtypeStruct((_N, 1, _B), jnp.int32),
    )(xt)


_mesh = plsc.VectorSubcoreMesh(core_axis_name="c", subcore_axis_name="s")


@functools.partial(
    pl.kernel,
    out_type=jax.ShapeDtypeStruct((_BN, _D), jnp.float32),
    mesh=_mesh,
    scratch_types=[
        pltpu.VMEM((_PER_W,), jnp.int32),
        pltpu.VMEM((_NBUF, _CH, _D), jnp.float32),
    ]
    + [pltpu.SemaphoreType.DMA] * _NBUF,
)
def _gather(idx_hbm, table_hbm, out_hbm, idx_v, rows_v, *sems):
    wid = lax.axis_index("s") * 2 + lax.axis_index("c")
    base = wid * _PER_W
    pltpu.sync_copy(idx_hbm.at[pl.ds(base, _PER_W)], idx_v)
    cps = [None] * _NBUF
    for c, (off, sz) in enumerate(_CHUNKS[: _NBUF - 1]):
        cps[c] = pltpu.async_copy(
            table_hbm.at[idx_v.at[pl.ds(off, sz)]],
            rows_v.at[c, pl.ds(0, sz)],
            sems[c],
        )
    for c, (off, sz) in enumerate(_CHUNKS):
        cur = c % _NBUF
        nxt = c + _NBUF - 1
        if nxt < len(_CHUNKS):
            noff, nsz = _CHUNKS[nxt]
            cps[nxt % _NBUF] = pltpu.async_copy(
                table_hbm.at[idx_v.at[pl.ds(noff, nsz)]],
                rows_v.at[nxt % _NBUF, pl.ds(0, nsz)],
                sems[nxt % _NBUF],
            )
        cps[cur].wait()
        pltpu.sync_copy(
            rows_v.at[cur, pl.ds(0, sz)], out_hbm.at[pl.ds(base + off, sz)]
        )


def kernel(x, table):
    xt = x.transpose(1, 2, 0)
    tokens = _argmax(xt).reshape(_BN)
    out = _gather(tokens, table)
    return out.reshape(_N, _B, _D).transpose(1, 0, 2)
